# baseline (speedup 1.0000x reference)
import jax
import jax.numpy as jnp
from jax import lax
from jax.experimental import pallas as pl
from jax.experimental.pallas import tpu as pltpu
from jax.experimental.pallas import tpu_sc as plsc

_NC = 2
_NS = 16
_NW = _NC * _NS

_B = 4096 * 200
_D = 128
_L = 128
_K = 2
_G = _K * _L
_IDX_ROWS = _B // _L
_ROWS_PER_W = _IDX_ROWS // _NW
_GROUPS = _ROWS_PER_W // _K
_PAIRS = _GROUPS // 2


def _emb_body(idx_hbm, table_hbm, out_hbm,
              idx_all, rows0, rows1, table_sp, gsem0, gsem1, ssem0, ssem1):
    rows = (rows0, rows1)
    gsems = (gsem0, gsem1)
    ssems = (ssem0, ssem1)

    sid = lax.axis_index("s")
    wid = sid * _NC + lax.axis_index("c")
    idx_row0 = wid * _ROWS_PER_W
    out_row0 = idx_row0 * _L

    @pl.when(sid == 0)
    def _stage():
        pltpu.sync_copy(table_hbm, table_sp)

    pltpu.sync_copy(idx_hbm.at[pl.ds(idx_row0, _ROWS_PER_W)], idx_all)

    plsc.subcore_barrier()

    def fetch(b, g):
        for j in range(_K):
            pltpu.async_copy(
                table_sp.at[idx_all.at[g * _K + j]],
                rows[b].at[pl.ds(j * _L, _L)],
                gsems[b],
            )

    def wait_gathers(b, g):
        for j in range(_K):
            pltpu.make_async_copy(
                table_sp.at[idx_all.at[g * _K + j]],
                rows[b].at[pl.ds(j * _L, _L)],
                gsems[b],
            ).wait()

    def fire_store(b, g):
        return pltpu.async_copy(
            rows[b], out_hbm.at[pl.ds(out_row0 + g * _G, _G)], ssems[b])

    fetch(0, 0)
    fetch(1, 1)

    def step(t, carry):
        g = 2 * t
        wait_gathers(0, g)
        s0 = fire_store(0, g)
        wait_gathers(1, g + 1)
        s1 = fire_store(1, g + 1)
        s0.wait()
        fetch(0, g + 2)
        s1.wait()
        fetch(1, g + 3)
        return carry

    lax.fori_loop(0, _PAIRS - 1, step, 0)
    g = 2 * (_PAIRS - 1)
    wait_gathers(0, g)
    s0 = fire_store(0, g)
    wait_gathers(1, g + 1)
    s1 = fire_store(1, g + 1)
    s0.wait()
    s1.wait()


_emb_kernel = pl.kernel(
    _emb_body,
    out_type=jax.ShapeDtypeStruct((_B, _D), jnp.float32),
    mesh=plsc.VectorSubcoreMesh(
        core_axis_name="c", subcore_axis_name="s",
        num_cores=_NC, num_subcores=_NS,
    ),
    scratch_types=[
        pltpu.VMEM((_ROWS_PER_W, _L), jnp.int32),
        pltpu.VMEM((_G, _D), jnp.float32),
        pltpu.VMEM((_G, _D), jnp.float32),
        pltpu.VMEM_SHARED((1001, _D), jnp.float32),
        pltpu.SemaphoreType.DMA,
        pltpu.SemaphoreType.DMA,
        pltpu.SemaphoreType.DMA,
        pltpu.SemaphoreType.DMA,
    ],
)


@jax.jit
def kernel(time_features, table):
    bsz, seq = time_features.shape
    idx = time_features.reshape(_IDX_ROWS, _L).astype(jnp.int32)
    table = table.at[0].set(0.0)
    out = _emb_kernel(idx, table)
    return out.reshape(bsz, seq, _D)


# speedup vs baseline: 1.3880x; 1.3880x over previous
import jax
import jax.numpy as jnp
from jax import lax
from jax.experimental import pallas as pl
from jax.experimental.pallas import tpu as pltpu
from jax.experimental.pallas import tpu_sc as plsc

_NC = 2
_NS = 16
_NW = _NC * _NS

_B = 4096 * 200
_D = 128
_V = 1001
_PER_W = _B // _NW
_G = 160
_K = 2
_LG = _G // _K
_GROUPS = _PER_W // _G
_NT = _GROUPS // 3
_LEFT = _GROUPS - 3 * _NT


def _emb_body(idx_hbm, table_hbm, out_hbm,
              idx_all, rows0, rows1, rows2, table_sp,
              gsem0, gsem1, gsem2, ssem0, ssem1, ssem2):
    rows = (rows0, rows1, rows2)
    gsems = (gsem0, gsem1, gsem2)
    ssems = (ssem0, ssem1, ssem2)

    sid = lax.axis_index("s")
    wid = sid * _NC + lax.axis_index("c")
    base = wid * _PER_W

    @pl.when(sid == 0)
    def _stage():
        pltpu.sync_copy(table_hbm, table_sp)

    pltpu.sync_copy(idx_hbm.at[pl.ds(base, _PER_W)], idx_all)

    plsc.subcore_barrier()

    def fetch(b, g):
        for j in range(_K):
            pltpu.async_copy(
                table_sp.at[idx_all.at[pl.ds(g * _G + j * _LG, _LG)]],
                rows[b].at[pl.ds(j * _LG, _LG)],
                gsems[b],
            )

    def wait_gathers(b, g):
        for j in range(_K):
            pltpu.make_async_copy(
                table_sp.at[idx_all.at[pl.ds(g * _G + j * _LG, _LG)]],
                rows[b].at[pl.ds(j * _LG, _LG)],
                gsems[b],
            ).wait()

    def fire_store(b, g):
        pltpu.async_copy(
            rows[b], out_hbm.at[pl.ds(base + g * _G, _G)], ssems[b])

    def drain_store(b, g):
        pltpu.make_async_copy(
            rows[b], out_hbm.at[pl.ds(base + g * _G, _G)], ssems[b]
        ).wait()

    def stage(g, b, bn, drain, do_fetch=True):
        wait_gathers(b, g)
        fire_store(b, g)
        if drain:
            drain_store(bn, g - 2)
        if do_fetch:
            fetch(bn, g + 1)

    fetch(0, 0)
    stage(0, 0, 1, False)
    stage(1, 1, 2, False)
    stage(2, 2, 0, True)

    def step(t, carry):
        g = 3 * t
        stage(g, 0, 1, True)
        stage(g + 1, 1, 2, True)
        stage(g + 2, 2, 0, True)
        return carry

    lax.fori_loop(1, _NT, step, 0)

    for i in range(_LEFT):
        g = 3 * _NT + i
        stage(g, g % 3, (g + 1) % 3, True, do_fetch=(g + 1 < _GROUPS))
    drain_store((_GROUPS - 2) % 3, _GROUPS - 2)
    drain_store((_GROUPS - 1) % 3, _GROUPS - 1)


_emb_kernel = pl.kernel(
    _emb_body,
    out_type=jax.ShapeDtypeStruct((_B, _D), jnp.float32),
    mesh=plsc.VectorSubcoreMesh(
        core_axis_name="c", subcore_axis_name="s",
        num_cores=_NC, num_subcores=_NS,
    ),
    scratch_types=[
        pltpu.VMEM((_PER_W,), jnp.int32),
        pltpu.VMEM((_G, _D), jnp.float32),
        pltpu.VMEM((_G, _D), jnp.float32),
        pltpu.VMEM((_G, _D), jnp.float32),
        pltpu.VMEM_SHARED((_V, _D), jnp.float32),
        pltpu.SemaphoreType.DMA,
        pltpu.SemaphoreType.DMA,
        pltpu.SemaphoreType.DMA,
        pltpu.SemaphoreType.DMA,
        pltpu.SemaphoreType.DMA,
        pltpu.SemaphoreType.DMA,
    ],
)


@jax.jit
def kernel(time_features, table):
    bsz, seq = time_features.shape
    idx = time_features.reshape(_B).astype(jnp.int32)
    table = table.at[0].set(0.0)
    out = _emb_kernel(idx, table)
    return out.reshape(bsz, seq, _D)


# speedup vs baseline: 1.4297x; 1.0300x over previous
import jax
import jax.numpy as jnp
from jax import lax
from jax.experimental import pallas as pl
from jax.experimental.pallas import tpu as pltpu
from jax.experimental.pallas import tpu_sc as plsc

_NC = 2
_NS = 16
_NW = _NC * _NS

_B = 4096 * 200
_D = 128
_K = 4
_LG = 80
_G = _K * _LG
_PER_W = _B // _NW
_GROUPS = _PER_W // _G
_PAIRS = _GROUPS // 2


def _emb_body(idx_hbm, table_hbm, out_hbm,
              idx_all, rows0, rows1, table_sp, gsem0, gsem1):
    rows = (rows0, rows1)
    gsems = (gsem0, gsem1)

    sid = lax.axis_index("s")
    wid = sid * _NC + lax.axis_index("c")
    base = wid * _PER_W

    @pl.when(sid == 0)
    def _stage():
        pltpu.sync_copy(table_hbm, table_sp)

    pltpu.sync_copy(idx_hbm.at[pl.ds(base, _PER_W)], idx_all)

    plsc.subcore_barrier()

    def fetch(b, g):
        for j in range(_K):
            pltpu.async_copy(
                table_sp.at[idx_all.at[pl.ds(g * _G + j * _LG, _LG)]],
                rows[b].at[pl.ds(j * _LG, _LG)],
                gsems[b],
            )

    def drain_store(b, g):
        for j in range(_K):
            pltpu.make_async_copy(
                table_sp.at[idx_all.at[pl.ds(g * _G + j * _LG, _LG)]],
                rows[b].at[pl.ds(j * _LG, _LG)],
                gsems[b],
            ).wait()
        pltpu.sync_copy(rows[b], out_hbm.at[pl.ds(base + g * _G, _G)])

    fetch(0, 0)
    fetch(1, 1)

    def step(t, carry):
        drain_store(0, 2 * t)
        fetch(0, 2 * t + 2)
        drain_store(1, 2 * t + 1)
        fetch(1, 2 * t + 3)
        return carry

    lax.fori_loop(0, _PAIRS - 1, step, 0)
    drain_store(0, 2 * (_PAIRS - 1))
    drain_store(1, 2 * (_PAIRS - 1) + 1)


_emb_kernel = pl.kernel(
    _emb_body,
    out_type=jax.ShapeDtypeStruct((_B, _D), jnp.float32),
    mesh=plsc.VectorSubcoreMesh(
        core_axis_name="c", subcore_axis_name="s",
        num_cores=_NC, num_subcores=_NS,
    ),
    scratch_types=[
        pltpu.VMEM((_PER_W,), jnp.int32),
        pltpu.VMEM((_G, _D), jnp.float32),
        pltpu.VMEM((_G, _D), jnp.float32),
        pltpu.VMEM_SHARED((1001, _D), jnp.float32),
        pltpu.SemaphoreType.DMA,
        pltpu.SemaphoreType.DMA,
    ],
)


@jax.jit
def kernel(time_features, table):
    bsz, seq = time_features.shape
    idx = time_features.reshape(_B).astype(jnp.int32)
    table = table.at[0].set(0.0)
    out = _emb_kernel(idx, table)
    return out.reshape(bsz, seq, _D)
